# baseline (speedup 1.0000x reference)
import numpy as np
import jax
import jax.numpy as jnp
from jax.experimental import pallas as pl
from jax.experimental.pallas import tpu as pltpu

_CB = 8192
_D = 32
_TR = 128
_NT = _CB // _TR

_K1 = np.uint32(2247515013)
_K2 = np.uint32(2545468385)

_R0 = (13, 15, 26, 6)
_R1 = (17, 29, 16, 24)


def _rotl(x, r):
    return (x << np.uint32(r)) | (x >> np.uint32(32 - r))


def _threefry_bits(p):
    ks0 = _K1
    ks1 = _K2
    ks2 = np.uint32(_K1 ^ _K2 ^ np.uint32(0x1BD11BDA))
    x0 = jnp.zeros_like(p) + ks0
    x1 = p + ks1

    def rounds(x0, x1, rots):
        for r in rots:
            x0 = x0 + x1
            x1 = _rotl(x1, r)
            x1 = x0 ^ x1
        return x0, x1

    x0, x1 = rounds(x0, x1, _R0)
    x0 = x0 + ks1
    x1 = x1 + np.uint32(ks2 + np.uint32(1))
    x0, x1 = rounds(x0, x1, _R1)
    x0 = x0 + ks2
    x1 = x1 + np.uint32(ks0 + np.uint32(2))
    x0, x1 = rounds(x0, x1, _R0)
    x0 = x0 + ks0
    x1 = x1 + np.uint32(ks1 + np.uint32(3))
    x0, x1 = rounds(x0, x1, _R1)
    x0 = x0 + ks1
    x1 = x1 + np.uint32(ks2 + np.uint32(4))
    x0, x1 = rounds(x0, x1, _R0)
    x0 = x0 + ks2
    x1 = x1 + np.uint32(ks0 + np.uint32(5))
    return x0 ^ x1


def _vq_kernel(x_ref, x2_ref, c2_ref, cbt_ref, cb_ref, codes_ref, part_ref):
    i = pl.program_id(0)
    xb = x_ref[...]
    x2 = x2_ref[...]
    c2 = c2_ref[...]
    cbt = cbt_ref[...]

    m = jnp.dot(xb, cbt, preferred_element_type=jnp.float32)
    d2 = (x2 + c2) - np.float32(2.0) * m
    d = jnp.sqrt(jnp.maximum(d2, np.float32(0.0)))

    rows = jax.lax.broadcasted_iota(jnp.int32, (_TR, _CB), 0)
    cols = jax.lax.broadcasted_iota(jnp.int32, (_TR, _CB), 1)
    p = ((i * _TR + rows) * _CB + cols).astype(jnp.uint32)
    bits = _threefry_bits(p)
    fb = (bits >> np.uint32(9)) | np.uint32(0x3F800000)
    f = jax.lax.bitcast_convert_type(fb, jnp.float32) - np.float32(1.0)
    mn = np.float32(1e-9)
    u = jnp.maximum(mn, f * (np.float32(1.0) - mn) + mn)
    g = -jnp.log(-jnp.log(u))

    z = -d + g
    rowmax = jnp.max(z, axis=1, keepdims=True)
    idx = jnp.min(jnp.where(z == rowmax, cols, _CB), axis=1, keepdims=True)
    onehot = (cols == idx).astype(jnp.float32)
    codes_ref[...] = onehot

    q = jnp.dot(onehot, cb_ref[...], preferred_element_type=jnp.float32,
                precision=jax.lax.Precision.HIGHEST)
    s = jnp.sum((q - xb) ** 2)
    part_ref[...] = (s * np.float32(1.25 / (_CB * _D))).reshape(1, 1, 1)


def kernel(inputs, codebook):
    x = inputs.reshape(-1, _D)
    x2 = jnp.sum(x * x, axis=1, keepdims=True)
    c2 = jnp.sum(codebook * codebook, axis=1)[None, :]
    cbt = codebook.T

    codes, parts = pl.pallas_call(
        _vq_kernel,
        grid=(_NT,),
        in_specs=[
            pl.BlockSpec((_TR, _D), lambda i: (i, 0)),
            pl.BlockSpec((_TR, 1), lambda i: (i, 0)),
            pl.BlockSpec((1, _CB), lambda i: (0, 0)),
            pl.BlockSpec((_D, _CB), lambda i: (0, 0)),
            pl.BlockSpec((_CB, _D), lambda i: (0, 0)),
        ],
        out_specs=[
            pl.BlockSpec((_TR, _CB), lambda i: (i, 0)),
            pl.BlockSpec((1, 1, 1), lambda i: (i, 0, 0)),
        ],
        out_shape=[
            jax.ShapeDtypeStruct((_CB, _CB), jnp.float32),
            jax.ShapeDtypeStruct((_NT, 1, 1), jnp.float32),
        ],
        compiler_params=pltpu.CompilerParams(
            dimension_semantics=("parallel",)),
    )(x, x2, c2, cbt, codebook)

    loss = jnp.sum(parts).reshape(())
    return codes, loss


# speedup vs baseline: 1.0044x; 1.0044x over previous
import numpy as np
import jax
import jax.numpy as jnp
from jax.experimental import pallas as pl
from jax.experimental.pallas import tpu as pltpu

_CB = 8192
_D = 32
_TR = 128
_NT = _CB // _TR

_K1 = np.uint32(2247515013)
_K2 = np.uint32(2545468385)

_R0 = (13, 15, 26, 6)
_R1 = (17, 29, 16, 24)


def _rotl(x, r):
    return (x << np.uint32(r)) | (x >> np.uint32(32 - r))


def _threefry_bits(p):
    ks0 = _K1
    ks1 = _K2
    ks2 = np.uint32(_K1 ^ _K2 ^ np.uint32(0x1BD11BDA))
    x0 = jnp.zeros_like(p) + ks0
    x1 = p + ks1

    def rounds(x0, x1, rots):
        for r in rots:
            x0 = x0 + x1
            x1 = _rotl(x1, r)
            x1 = x0 ^ x1
        return x0, x1

    x0, x1 = rounds(x0, x1, _R0)
    x0 = x0 + ks1
    x1 = x1 + np.uint32(ks2 + np.uint32(1))
    x0, x1 = rounds(x0, x1, _R1)
    x0 = x0 + ks2
    x1 = x1 + np.uint32(ks0 + np.uint32(2))
    x0, x1 = rounds(x0, x1, _R0)
    x0 = x0 + ks0
    x1 = x1 + np.uint32(ks1 + np.uint32(3))
    x0, x1 = rounds(x0, x1, _R1)
    x0 = x0 + ks1
    x1 = x1 + np.uint32(ks2 + np.uint32(4))
    x0, x1 = rounds(x0, x1, _R0)
    x0 = x0 + ks2
    x1 = x1 + np.uint32(ks0 + np.uint32(5))
    return x0 ^ x1


def _vq_kernel(x_ref, x2_ref, c2_ref, cbt_ref, cb_ref, codes_ref, part_ref):
    i = pl.program_id(0)
    xb = x_ref[...]
    x2 = x2_ref[...]
    c2 = c2_ref[...]
    cbt = cbt_ref[...]

    m = jnp.dot(xb, cbt, preferred_element_type=jnp.float32)
    d2 = (x2 + c2) - np.float32(2.0) * m
    d = jnp.sqrt(jnp.maximum(d2, np.float32(0.0)))

    rows = jax.lax.broadcasted_iota(jnp.int32, (_TR, _CB), 0)
    cols = jax.lax.broadcasted_iota(jnp.int32, (_TR, _CB), 1)
    p = ((i * _TR + rows) * _CB + cols).astype(jnp.uint32)
    bits = _threefry_bits(p)
    fb = (bits >> np.uint32(9)) | np.uint32(0x3F800000)
    f = jax.lax.bitcast_convert_type(fb, jnp.float32) - np.float32(1.0)
    u = f + np.float32(1e-9)
    g = -jnp.log(-jnp.log(u))

    z = -d + g
    rowmax = jnp.max(z, axis=1, keepdims=True)
    idx = jnp.min(jnp.where(z == rowmax, cols, _CB), axis=1, keepdims=True)
    onehot = (cols == idx).astype(jnp.float32)
    codes_ref[...] = onehot

    q = jnp.dot(onehot, cb_ref[...], preferred_element_type=jnp.float32,
                precision=jax.lax.Precision.HIGHEST)
    s = jnp.sum((q - xb) ** 2)
    part_ref[...] = (s * np.float32(1.25 / (_CB * _D))).reshape(1, 1, 1)


def kernel(inputs, codebook):
    x = inputs.reshape(-1, _D)
    x2 = jnp.sum(x * x, axis=1, keepdims=True)
    c2 = jnp.sum(codebook * codebook, axis=1)[None, :]
    cbt = codebook.T

    codes, parts = pl.pallas_call(
        _vq_kernel,
        grid=(_NT,),
        in_specs=[
            pl.BlockSpec((_TR, _D), lambda i: (i, 0)),
            pl.BlockSpec((_TR, 1), lambda i: (i, 0)),
            pl.BlockSpec((1, _CB), lambda i: (0, 0)),
            pl.BlockSpec((_D, _CB), lambda i: (0, 0)),
            pl.BlockSpec((_CB, _D), lambda i: (0, 0)),
        ],
        out_specs=[
            pl.BlockSpec((_TR, _CB), lambda i: (i, 0)),
            pl.BlockSpec((1, 1, 1), lambda i: (i, 0, 0)),
        ],
        out_shape=[
            jax.ShapeDtypeStruct((_CB, _CB), jnp.float32),
            jax.ShapeDtypeStruct((_NT, 1, 1), jnp.float32),
        ],
        compiler_params=pltpu.CompilerParams(
            dimension_semantics=("parallel",)),
    )(x, x2, c2, cbt, codebook)

    loss = jnp.sum(parts).reshape(())
    return codes, loss


# speedup vs baseline: 1.0084x; 1.0040x over previous
import numpy as np
import jax
import jax.numpy as jnp
from jax.experimental import pallas as pl
from jax.experimental.pallas import tpu as pltpu

_CB = 8192
_D = 32
_TR = 128
_NT = _CB // _TR

_K1 = np.uint32(2247515013)
_K2 = np.uint32(2545468385)

_R0 = (13, 15, 26, 6)
_R1 = (17, 29, 16, 24)


def _rotl(x, r):
    return (x << np.uint32(r)) | (x >> np.uint32(32 - r))


def _threefry_bits(p):
    ks0 = _K1
    ks1 = _K2
    ks2 = np.uint32(_K1 ^ _K2 ^ np.uint32(0x1BD11BDA))
    x0 = jnp.zeros_like(p) + ks0
    x1 = p + ks1

    def rounds(x0, x1, rots):
        for r in rots:
            x0 = x0 + x1
            x1 = _rotl(x1, r)
            x1 = x0 ^ x1
        return x0, x1

    x0, x1 = rounds(x0, x1, _R0)
    x0 = x0 + ks1
    x1 = x1 + np.uint32(ks2 + np.uint32(1))
    x0, x1 = rounds(x0, x1, _R1)
    x0 = x0 + ks2
    x1 = x1 + np.uint32(ks0 + np.uint32(2))
    x0, x1 = rounds(x0, x1, _R0)
    x0 = x0 + ks0
    x1 = x1 + np.uint32(ks1 + np.uint32(3))
    x0, x1 = rounds(x0, x1, _R1)
    x0 = x0 + ks1
    x1 = x1 + np.uint32(ks2 + np.uint32(4))
    x0, x1 = rounds(x0, x1, _R0)
    x0 = x0 + ks2
    x1 = x1 + np.uint32(ks0 + np.uint32(5))
    return x0 ^ x1


def _vq_kernel(lin_ref, x_ref, x2_ref, c2_ref, cbt_ref, cb_ref,
               codes_ref, part_ref):
    i = pl.program_id(0)
    xb = x_ref[...]
    x2 = x2_ref[...]
    c2 = c2_ref[...]
    cbt = cbt_ref[...]

    m = jnp.dot(xb, cbt, preferred_element_type=jnp.float32)
    d2 = (x2 + c2) - np.float32(2.0) * m
    d = jnp.sqrt(jnp.maximum(d2, np.float32(0.0)))

    cols = jax.lax.broadcasted_iota(jnp.int32, (_TR, _CB), 1)
    p = lin_ref[...] + np.uint32(_TR * _CB) * i.astype(jnp.uint32)
    bits = _threefry_bits(p)
    fb = (bits >> np.uint32(9)) | np.uint32(0x3F800000)
    f = jax.lax.bitcast_convert_type(fb, jnp.float32) - np.float32(1.0)
    u = f + np.float32(1e-9)
    g = -jnp.log(-jnp.log(u))

    z = g - d
    rowmax = jnp.max(z, axis=1, keepdims=True)
    idx = jnp.min(jnp.where(z == rowmax, cols, _CB), axis=1, keepdims=True)
    onehot = (cols == idx).astype(jnp.float32)
    codes_ref[...] = onehot

    q = jnp.dot(onehot, cb_ref[...], preferred_element_type=jnp.float32,
                precision=jax.lax.Precision.HIGHEST)
    s = jnp.sum((q - xb) ** 2)
    part_ref[...] = (s * np.float32(1.25 / (_CB * _D))).reshape(1, 1, 1)


def kernel(inputs, codebook):
    x = inputs.reshape(-1, _D)
    x2 = jnp.sum(x * x, axis=1, keepdims=True)
    c2 = jnp.sum(codebook * codebook, axis=1)[None, :]
    cbt = codebook.T
    lin = jax.lax.broadcasted_iota(jnp.uint32, (_TR, _CB), 0) * np.uint32(_CB) \
        + jax.lax.broadcasted_iota(jnp.uint32, (_TR, _CB), 1)

    codes, parts = pl.pallas_call(
        _vq_kernel,
        grid=(_NT,),
        in_specs=[
            pl.BlockSpec((_TR, _CB), lambda i: (0, 0)),
            pl.BlockSpec((_TR, _D), lambda i: (i, 0)),
            pl.BlockSpec((_TR, 1), lambda i: (i, 0)),
            pl.BlockSpec((1, _CB), lambda i: (0, 0)),
            pl.BlockSpec((_D, _CB), lambda i: (0, 0)),
            pl.BlockSpec((_CB, _D), lambda i: (0, 0)),
        ],
        out_specs=[
            pl.BlockSpec((_TR, _CB), lambda i: (i, 0)),
            pl.BlockSpec((1, 1, 1), lambda i: (i, 0, 0)),
        ],
        out_shape=[
            jax.ShapeDtypeStruct((_CB, _CB), jnp.float32),
            jax.ShapeDtypeStruct((_NT, 1, 1), jnp.float32),
        ],
        compiler_params=pltpu.CompilerParams(
            dimension_semantics=("parallel",)),
    )(lin, x, x2, c2, cbt, codebook)

    loss = jnp.sum(parts).reshape(())
    return codes, loss
